# baseline (speedup 1.0000x reference)
import functools

import jax
import jax.numpy as jnp
from jax import lax
from jax.experimental import pallas as pl
from jax.experimental.pallas import tpu as pltpu
from jax.experimental.pallas import tpu_sc as plsc

N = 10000
E = 160000
D_IN = 256
H1 = 512
LAT = 128
Q = 2
QW = 128
NC, NS = 2, 16

K = 128
CH_P = 80
NPH = 2
HCH = CH_P // NPH
EP_P = NS * CH_P * K
NPAD = 10112
NROW = NPAD // NS
DUMMY = 10048

KD = 128
CH_D = 40
EP_D = NC * NS * CH_D * KD
NPAD_D = 10240
NROW_D = NPAD_D // NS
DUMMY_D = 10016

RB = 512
GRID = (N + RB - 1) // RB


def _deg_body(dst_hbm, zeros_hbm, ones_hbm, out_hbm, idx_v, ones_v, acc_s):
    c = lax.axis_index("c")
    s = lax.axis_index("s")
    pltpu.sync_copy(zeros_hbm.at[pl.ds(s * NROW_D, NROW_D)],
                    acc_s.at[pl.ds(s * NROW_D, NROW_D)])
    pltpu.sync_copy(dst_hbm.at[c, s], idx_v)
    pltpu.sync_copy(ones_hbm, ones_v)
    plsc.subcore_barrier()

    def step(j, carry):
        pltpu.sync_copy(ones_v, acc_s.at[idx_v.at[j]], add=True)
        return carry

    lax.fori_loop(0, CH_D, step, 0)
    plsc.subcore_barrier()
    pltpu.sync_copy(acc_s.at[pl.ds(s * NROW_D, NROW_D)],
                    out_hbm.at[c, pl.ds(s * NROW_D, NROW_D)])


@functools.cache
def _deg_kernel():
    mesh = plsc.VectorSubcoreMesh(core_axis_name="c", subcore_axis_name="s",
                                  num_cores=NC, num_subcores=NS)
    return pl.kernel(
        _deg_body,
        out_type=jax.ShapeDtypeStruct((NC, NPAD_D), jnp.float32),
        mesh=mesh,
        scratch_types=[
            pltpu.VMEM((CH_D, KD), jnp.int32),
            pltpu.VMEM((KD,), jnp.float32),
            pltpu.VMEM_SHARED((NPAD_D,), jnp.float32),
        ],
    )


def _prop_body(u_hbm, src_hbm, dst_hbm, zeros_hbm, out_hbm,
               src_v, dst_v, rows0_v, rows1_v, acc_s, g0, g1):
    c = lax.axis_index("c")
    s = lax.axis_index("s")
    pltpu.sync_copy(zeros_hbm.at[pl.ds(s * NROW, NROW)],
                    acc_s.at[pl.ds(s * NROW, NROW)])
    plsc.subcore_barrier()

    def phase(h, hcarry):
        pltpu.sync_copy(src_hbm.at[(c * NS + s) * NPH + h], src_v)
        pltpu.sync_copy(dst_hbm.at[s * NPH + h], dst_v)

        def step(i, carry):
            j = 2 * i
            d0 = pltpu.async_copy(u_hbm.at[src_v.at[j]], rows0_v, g0)
            d1 = pltpu.async_copy(u_hbm.at[src_v.at[j + 1]], rows1_v, g1)
            d0.wait()
            pltpu.sync_copy(rows0_v, acc_s.at[dst_v.at[j]], add=True)
            d1.wait()
            pltpu.sync_copy(rows1_v, acc_s.at[dst_v.at[j + 1]], add=True)
            return carry

        lax.fori_loop(0, HCH // 2, step, 0)
        return hcarry

    lax.fori_loop(0, NPH, phase, 0)
    plsc.subcore_barrier()
    pltpu.sync_copy(acc_s.at[pl.ds(s * NROW, NROW)],
                    out_hbm.at[pl.ds(c * NPAD + s * NROW, NROW)])


@functools.cache
def _prop_kernel():
    mesh = plsc.VectorSubcoreMesh(core_axis_name="c", subcore_axis_name="s",
                                  num_cores=NC, num_subcores=NS)
    return pl.kernel(
        _prop_body,
        out_type=jax.ShapeDtypeStruct((Q * NPAD, QW), jnp.float32),
        mesh=mesh,
        scratch_types=[
            pltpu.VMEM((HCH, K), jnp.int32),
            pltpu.VMEM((HCH, K), jnp.int32),
            pltpu.VMEM((K, QW), jnp.float32),
            pltpu.VMEM((K, QW), jnp.float32),
            pltpu.VMEM_SHARED((NPAD, QW), jnp.float32),
            pltpu.SemaphoreType.DMA,
            pltpu.SemaphoreType.DMA,
        ],
    )


def _quarters(t):
    return jnp.stack([t[:, i * QW:(i + 1) * QW] for i in range(Q)], axis=0)


def _pre_body(deg_ref, x_ref, dis_ref, xs_ref):
    deg = deg_ref[0, :] + deg_ref[1, :] + 1.0
    dis = lax.rsqrt(deg)
    dis_ref[...] = dis[:, None]
    xs_ref[...] = _quarters(x_ref[...] * dis[:, None])


def _mid1_body(p_ref, u_ref, dis_ref, x_ref, n_ref,
               w1_ref, b1_ref, w2a_ref, w2b_ref, w2c_ref, out_ref):
    dis = dis_ref[...]
    y1 = jnp.concatenate([p_ref[i] + u_ref[i] for i in range(Q)], axis=1) * dis
    h1 = jnp.dot(y1, w1_ref[...], preferred_element_type=jnp.float32)
    h1 = jax.nn.relu(h1 + b1_ref[...])
    t2 = jnp.dot(x_ref[...], w2a_ref[...], preferred_element_type=jnp.float32)
    t2 += jnp.dot(n_ref[...], w2b_ref[...], preferred_element_type=jnp.float32)
    t2 += jnp.dot(h1, w2c_ref[...], preferred_element_type=jnp.float32)
    out_ref[...] = _quarters(t2 * dis)


def _mid2_body(p_ref, u_ref, dis_ref, x_ref,
               b2_ref, w3a_ref, w3b_ref, out_ref):
    dis = dis_ref[...]
    h2 = jnp.concatenate([p_ref[i] + u_ref[i] for i in range(Q)], axis=1) * dis
    h2 = jax.nn.relu(h2 + b2_ref[...])
    t3 = jnp.dot(x_ref[...], w3a_ref[...], preferred_element_type=jnp.float32)
    t3 += jnp.dot(h2, w3b_ref[...], preferred_element_type=jnp.float32)
    out_ref[...] = _quarters(t3 * dis)


def _post_body(p_ref, u_ref, dis_ref, bm_ref, bl_ref, zm_ref, zl_ref):
    dis = dis_ref[...]
    z = jnp.concatenate([p_ref[i] + u_ref[i] for i in range(Q)], axis=1) * dis
    zm_ref[...] = z[:, :LAT] + bm_ref[...]
    zl_ref[...] = z[:, LAT:] + bl_ref[...]


def _rowspec(width):
    return pl.BlockSpec((RB, width), lambda i: (i, 0))


def _qspec():
    return pl.BlockSpec((Q, RB, QW), lambda i: (0, i, 0))


def _fullspec(shape):
    nd = len(shape)
    return pl.BlockSpec(shape, lambda i, _nd=nd: (0,) * _nd)


_pre_call = pl.pallas_call(
    _pre_body,
    grid=(GRID,),
    in_specs=[pl.BlockSpec((NC, RB), lambda i: (0, i)), _rowspec(D_IN)],
    out_specs=[_rowspec(1), _qspec()],
    out_shape=[jax.ShapeDtypeStruct((N, 1), jnp.float32),
               jax.ShapeDtypeStruct((Q, NPAD, QW), jnp.float32)],
)

_mid1_call = pl.pallas_call(
    _mid1_body,
    grid=(GRID,),
    in_specs=[_qspec(), _qspec(), _rowspec(1), _rowspec(D_IN),
              _rowspec(D_IN), _fullspec((D_IN, H1)),
              _fullspec((1, H1)), _fullspec((D_IN, D_IN)),
              _fullspec((D_IN, D_IN)), _fullspec((H1, D_IN))],
    out_specs=_qspec(),
    out_shape=jax.ShapeDtypeStruct((Q, NPAD, QW), jnp.float32),
)

_mid2_call = pl.pallas_call(
    _mid2_body,
    grid=(GRID,),
    in_specs=[_qspec(), _qspec(), _rowspec(1), _rowspec(D_IN),
              _fullspec((1, D_IN)), _fullspec((D_IN, D_IN)),
              _fullspec((D_IN, D_IN))],
    out_specs=_qspec(),
    out_shape=jax.ShapeDtypeStruct((Q, NPAD, QW), jnp.float32),
)

_post_call = pl.pallas_call(
    _post_body,
    grid=(GRID,),
    in_specs=[_qspec(), _qspec(), _rowspec(1),
              _fullspec((1, LAT)), _fullspec((1, LAT))],
    out_specs=[_rowspec(LAT), _rowspec(LAT)],
    out_shape=[jax.ShapeDtypeStruct((N, LAT), jnp.float32),
               jax.ShapeDtypeStruct((N, LAT), jnp.float32)],
)


def kernel(x, edge_index, noise, W1, b1, W2, b2, Wm, bm, Wl, bl):
    src = edge_index[0].astype(jnp.int32)
    dst = edge_index[1].astype(jnp.int32)

    dst_d = jnp.concatenate(
        [dst, jnp.full((EP_D - E,), DUMMY_D, jnp.int32)]).reshape(NC, NS, CH_D, KD)
    src_p = jnp.concatenate([src, jnp.zeros((EP_P - E,), jnp.int32)])
    src_p = jnp.stack([src_p, src_p + NPAD]).reshape(NC * NS * NPH, HCH, K)
    dst_p = jnp.concatenate(
        [dst, jnp.full((EP_P - E,), DUMMY, jnp.int32)]).reshape(NS * NPH, HCH, K)

    zeros_1d = jnp.zeros((NPAD_D,), jnp.float32)
    ones_k = jnp.ones((KD,), jnp.float32)
    zeros_q = jnp.zeros((NPAD, QW), jnp.float32)

    w2a, w2b, w2c = W2[:D_IN], W2[D_IN:2 * D_IN], W2[2 * D_IN:]
    W3 = jnp.concatenate([Wm, Wl], axis=1)
    w3a, w3b = W3[:D_IN], W3[D_IN:]
    b1r = b1.reshape(1, H1)
    b2r = b2.reshape(1, D_IN)
    bmr, blr = bm.reshape(1, LAT), bl.reshape(1, LAT)

    deg2 = _deg_kernel()(dst_d, zeros_1d, ones_k)

    prop = _prop_kernel()
    dis, xs4 = _pre_call(deg2, x)
    p1 = prop(xs4.reshape(Q * NPAD, QW), src_p, dst_p, zeros_q)
    t2s = _mid1_call(p1.reshape(Q, NPAD, QW), xs4, dis, x, noise,
                     W1, b1r, w2a, w2b, w2c)
    p2 = prop(t2s.reshape(Q * NPAD, QW), src_p, dst_p, zeros_q)
    t3s = _mid2_call(p2.reshape(Q, NPAD, QW), t2s, dis, x, b2r, w3a, w3b)
    p3 = prop(t3s.reshape(Q * NPAD, QW), src_p, dst_p, zeros_q)
    z_mean, z_logstd = _post_call(p3.reshape(Q, NPAD, QW), t3s, dis, bmr, blr)
    return (z_mean, z_logstd)


# speedup vs baseline: 1.1688x; 1.1688x over previous
import functools

import jax
import jax.numpy as jnp
from jax import lax
from jax.experimental import pallas as pl
from jax.experimental.pallas import tpu as pltpu
from jax.experimental.pallas import tpu_sc as plsc

N = 10000
E = 160000
D_IN = 256
H1 = 512
LAT = 128
Q = 2
QW = 128
NC, NS = 2, 16

K = 128
CH_P = 79
EP_P = NS * CH_P * K
NPAD = 10112
NROW = NPAD // NS
DUMMY = 10048

KD = 128
CH_D = 40
EP_D = NC * NS * CH_D * KD
NPAD_D = 10240
NROW_D = NPAD_D // NS
DUMMY_D = 10016

RB = 512
GRID = (N + RB - 1) // RB


def _deg_body(dst_hbm, zeros_hbm, ones_hbm, out_hbm, idx_v, ones_v, acc_s):
    c = lax.axis_index("c")
    s = lax.axis_index("s")
    pltpu.sync_copy(zeros_hbm.at[pl.ds(s * NROW_D, NROW_D)],
                    acc_s.at[pl.ds(s * NROW_D, NROW_D)])
    pltpu.sync_copy(dst_hbm.at[c, s], idx_v)
    pltpu.sync_copy(ones_hbm, ones_v)
    plsc.subcore_barrier()

    def step(j, carry):
        pltpu.sync_copy(ones_v, acc_s.at[idx_v.at[j]], add=True)
        return carry

    lax.fori_loop(0, CH_D, step, 0)
    plsc.subcore_barrier()
    pltpu.sync_copy(acc_s.at[pl.ds(s * NROW_D, NROW_D)],
                    out_hbm.at[c, pl.ds(s * NROW_D, NROW_D)])


@functools.cache
def _deg_kernel():
    mesh = plsc.VectorSubcoreMesh(core_axis_name="c", subcore_axis_name="s",
                                  num_cores=NC, num_subcores=NS)
    return pl.kernel(
        _deg_body,
        out_type=jax.ShapeDtypeStruct((NC, NPAD_D), jnp.float32),
        mesh=mesh,
        scratch_types=[
            pltpu.VMEM((CH_D, KD), jnp.int32),
            pltpu.VMEM((KD,), jnp.float32),
            pltpu.VMEM_SHARED((NPAD_D,), jnp.float32),
        ],
    )


def _prop_body(u_hbm, src_hbm, dst_hbm, zeros_hbm, out_hbm,
               src_v, dst_v, rows_v, acc_s, sem):
    c = lax.axis_index("c")
    s = lax.axis_index("s")
    pltpu.sync_copy(zeros_hbm.at[pl.ds(s * NROW, NROW)],
                    acc_s.at[pl.ds(s * NROW, NROW)])
    pltpu.sync_copy(src_hbm.at[c, s], src_v)
    pltpu.sync_copy(dst_hbm.at[s], dst_v)
    plsc.subcore_barrier()

    def step(j, carry):
        pltpu.async_copy(u_hbm.at[src_v.at[j]], rows_v, sem).wait()
        pltpu.sync_copy(rows_v, acc_s.at[dst_v.at[j]], add=True)
        return carry

    lax.fori_loop(0, CH_P, step, 0)
    plsc.subcore_barrier()
    pltpu.sync_copy(acc_s.at[pl.ds(s * NROW, NROW)],
                    out_hbm.at[pl.ds(c * NPAD + s * NROW, NROW)])


@functools.cache
def _prop_kernel():
    mesh = plsc.VectorSubcoreMesh(core_axis_name="c", subcore_axis_name="s",
                                  num_cores=NC, num_subcores=NS)
    return pl.kernel(
        _prop_body,
        out_type=jax.ShapeDtypeStruct((Q * NPAD, QW), jnp.float32),
        mesh=mesh,
        scratch_types=[
            pltpu.VMEM((CH_P, K), jnp.int32),
            pltpu.VMEM((CH_P, K), jnp.int32),
            pltpu.VMEM((K, QW), jnp.float32),
            pltpu.VMEM_SHARED((NPAD, QW), jnp.float32),
            pltpu.SemaphoreType.DMA,
        ],
    )


def _quarters(t):
    return jnp.stack([t[:, i * QW:(i + 1) * QW] for i in range(Q)], axis=0)


def _pre_body(deg_ref, x_ref, dis_ref, xs_ref):
    deg = deg_ref[0, :] + deg_ref[1, :] + 1.0
    dis = lax.rsqrt(deg)
    dis_ref[...] = dis[:, None]
    xs_ref[...] = _quarters(x_ref[...] * dis[:, None])


def _mid1_body(p_ref, u_ref, dis_ref, x_ref, n_ref,
               w1_ref, b1_ref, w2a_ref, w2b_ref, w2c_ref, out_ref):
    dis = dis_ref[...]
    y1 = jnp.concatenate([p_ref[i] + u_ref[i] for i in range(Q)], axis=1) * dis
    h1 = jnp.dot(y1, w1_ref[...], preferred_element_type=jnp.float32)
    h1 = jax.nn.relu(h1 + b1_ref[...])
    t2 = jnp.dot(x_ref[...], w2a_ref[...], preferred_element_type=jnp.float32)
    t2 += jnp.dot(n_ref[...], w2b_ref[...], preferred_element_type=jnp.float32)
    t2 += jnp.dot(h1, w2c_ref[...], preferred_element_type=jnp.float32)
    out_ref[...] = _quarters(t2 * dis)


def _mid2_body(p_ref, u_ref, dis_ref, x_ref,
               b2_ref, w3a_ref, w3b_ref, out_ref):
    dis = dis_ref[...]
    h2 = jnp.concatenate([p_ref[i] + u_ref[i] for i in range(Q)], axis=1) * dis
    h2 = jax.nn.relu(h2 + b2_ref[...])
    t3 = jnp.dot(x_ref[...], w3a_ref[...], preferred_element_type=jnp.float32)
    t3 += jnp.dot(h2, w3b_ref[...], preferred_element_type=jnp.float32)
    out_ref[...] = _quarters(t3 * dis)


def _post_body(p_ref, u_ref, dis_ref, bm_ref, bl_ref, zm_ref, zl_ref):
    dis = dis_ref[...]
    z = jnp.concatenate([p_ref[i] + u_ref[i] for i in range(Q)], axis=1) * dis
    zm_ref[...] = z[:, :LAT] + bm_ref[...]
    zl_ref[...] = z[:, LAT:] + bl_ref[...]


def _rowspec(width):
    return pl.BlockSpec((RB, width), lambda i: (i, 0))


def _qspec():
    return pl.BlockSpec((Q, RB, QW), lambda i: (0, i, 0))


def _fullspec(shape):
    nd = len(shape)
    return pl.BlockSpec(shape, lambda i, _nd=nd: (0,) * _nd)


_pre_call = pl.pallas_call(
    _pre_body,
    grid=(GRID,),
    in_specs=[pl.BlockSpec((NC, RB), lambda i: (0, i)), _rowspec(D_IN)],
    out_specs=[_rowspec(1), _qspec()],
    out_shape=[jax.ShapeDtypeStruct((N, 1), jnp.float32),
               jax.ShapeDtypeStruct((Q, N, QW), jnp.float32)],
)

_mid1_call = pl.pallas_call(
    _mid1_body,
    grid=(GRID,),
    in_specs=[_qspec(), _qspec(), _rowspec(1), _rowspec(D_IN),
              _rowspec(D_IN), _fullspec((D_IN, H1)),
              _fullspec((1, H1)), _fullspec((D_IN, D_IN)),
              _fullspec((D_IN, D_IN)), _fullspec((H1, D_IN))],
    out_specs=_qspec(),
    out_shape=jax.ShapeDtypeStruct((Q, N, QW), jnp.float32),
)

_mid2_call = pl.pallas_call(
    _mid2_body,
    grid=(GRID,),
    in_specs=[_qspec(), _qspec(), _rowspec(1), _rowspec(D_IN),
              _fullspec((1, D_IN)), _fullspec((D_IN, D_IN)),
              _fullspec((D_IN, D_IN))],
    out_specs=_qspec(),
    out_shape=jax.ShapeDtypeStruct((Q, N, QW), jnp.float32),
)

_post_call = pl.pallas_call(
    _post_body,
    grid=(GRID,),
    in_specs=[_qspec(), _qspec(), _rowspec(1),
              _fullspec((1, LAT)), _fullspec((1, LAT))],
    out_specs=[_rowspec(LAT), _rowspec(LAT)],
    out_shape=[jax.ShapeDtypeStruct((N, LAT), jnp.float32),
               jax.ShapeDtypeStruct((N, LAT), jnp.float32)],
)


def kernel(x, edge_index, noise, W1, b1, W2, b2, Wm, bm, Wl, bl):
    src = edge_index[0].astype(jnp.int32)
    dst = edge_index[1].astype(jnp.int32)

    dst_d = jnp.concatenate(
        [dst, jnp.full((EP_D - E,), DUMMY_D, jnp.int32)]).reshape(NC, NS, CH_D, KD)
    src_p = jnp.concatenate([src, jnp.zeros((EP_P - E,), jnp.int32)])
    src_p = jnp.stack([src_p, src_p + N]).reshape(NC, NS, CH_P, K)
    dst_p = jnp.concatenate(
        [dst, jnp.full((EP_P - E,), DUMMY, jnp.int32)]).reshape(NS, CH_P, K)

    zeros_1d = jnp.zeros((NPAD_D,), jnp.float32)
    ones_k = jnp.ones((KD,), jnp.float32)
    zeros_q = jnp.zeros((NPAD, QW), jnp.float32)

    w2a, w2b, w2c = W2[:D_IN], W2[D_IN:2 * D_IN], W2[2 * D_IN:]
    W3 = jnp.concatenate([Wm, Wl], axis=1)
    w3a, w3b = W3[:D_IN], W3[D_IN:]
    b1r = b1.reshape(1, H1)
    b2r = b2.reshape(1, D_IN)
    bmr, blr = bm.reshape(1, LAT), bl.reshape(1, LAT)

    deg2 = _deg_kernel()(dst_d, zeros_1d, ones_k)

    prop = _prop_kernel()
    dis, xs4 = _pre_call(deg2, x)
    p1 = prop(xs4.reshape(Q * N, QW), src_p, dst_p, zeros_q)
    t2s = _mid1_call(p1.reshape(Q, NPAD, QW), xs4, dis, x, noise,
                     W1, b1r, w2a, w2b, w2c)
    p2 = prop(t2s.reshape(Q * N, QW), src_p, dst_p, zeros_q)
    t3s = _mid2_call(p2.reshape(Q, NPAD, QW), t2s, dis, x, b2r, w3a, w3b)
    p3 = prop(t3s.reshape(Q * N, QW), src_p, dst_p, zeros_q)
    z_mean, z_logstd = _post_call(p3.reshape(Q, NPAD, QW), t3s, dis, bmr, blr)
    return (z_mean, z_logstd)


# speedup vs baseline: 1.2867x; 1.1008x over previous
import functools

import jax
import jax.numpy as jnp
from jax import lax
from jax.experimental import pallas as pl
from jax.experimental.pallas import tpu as pltpu
from jax.experimental.pallas import tpu_sc as plsc

N = 10000
E = 160000
D_IN = 256
H1 = 512
LAT = 128
Q = 2
QW = 128
NC, NS = 2, 16

K = 128
CH_P = 79
EP_P = NS * CH_P * K
NPAD = 10112
NROW = NPAD // NS
DUMMY = 10048

KD = 128
CH_D = 40
EP_D = NC * NS * CH_D * KD
NPAD_D = 10240
NROW_D = NPAD_D // NS
DUMMY_D = 10016

RB = 512
GRID = (N + RB - 1) // RB


def _deg_body(dst_hbm, zeros_hbm, ones_hbm, out_hbm, idx_v, ones_v, acc_s):
    c = lax.axis_index("c")
    s = lax.axis_index("s")
    pltpu.sync_copy(zeros_hbm.at[pl.ds(s * NROW_D, NROW_D)],
                    acc_s.at[pl.ds(s * NROW_D, NROW_D)])
    pltpu.sync_copy(dst_hbm.at[c, s], idx_v)
    pltpu.sync_copy(ones_hbm, ones_v)
    plsc.subcore_barrier()

    def step(j, carry):
        pltpu.sync_copy(ones_v, acc_s.at[idx_v.at[j]], add=True)
        return carry

    lax.fori_loop(0, CH_D, step, 0)
    plsc.subcore_barrier()
    pltpu.sync_copy(acc_s.at[pl.ds(s * NROW_D, NROW_D)],
                    out_hbm.at[c, pl.ds(s * NROW_D, NROW_D)])


@functools.cache
def _deg_kernel():
    mesh = plsc.VectorSubcoreMesh(core_axis_name="c", subcore_axis_name="s",
                                  num_cores=NC, num_subcores=NS)
    return pl.kernel(
        _deg_body,
        out_type=jax.ShapeDtypeStruct((NC, NPAD_D), jnp.float32),
        mesh=mesh,
        scratch_types=[
            pltpu.VMEM((CH_D, KD), jnp.int32),
            pltpu.VMEM((KD,), jnp.float32),
            pltpu.VMEM_SHARED((NPAD_D,), jnp.float32),
        ],
    )


def _prop_body(u_hbm, src_hbm, dst_hbm, zeros_hbm, out_hbm,
               src_v, dst_v, rows_v, acc_s, sem):
    c = lax.axis_index("c")
    s = lax.axis_index("s")
    pltpu.sync_copy(zeros_hbm.at[pl.ds(s * NROW, NROW)],
                    acc_s.at[pl.ds(s * NROW, NROW)])
    pltpu.sync_copy(src_hbm.at[c, s], src_v)
    pltpu.sync_copy(dst_hbm.at[s], dst_v)
    plsc.subcore_barrier()

    def step(j, carry):
        pltpu.async_copy(u_hbm.at[src_v.at[j]], rows_v, sem).wait()
        pltpu.sync_copy(rows_v, acc_s.at[dst_v.at[j]], add=True)
        return carry

    lax.fori_loop(0, CH_P, step, 0)
    plsc.subcore_barrier()
    pltpu.sync_copy(acc_s.at[pl.ds(s * NROW, NROW)],
                    out_hbm.at[pl.ds(c * NPAD + s * NROW, NROW)])


@functools.cache
def _prop_kernel():
    mesh = plsc.VectorSubcoreMesh(core_axis_name="c", subcore_axis_name="s",
                                  num_cores=NC, num_subcores=NS)
    return pl.kernel(
        _prop_body,
        out_type=jax.ShapeDtypeStruct((Q * NPAD, QW), jnp.float32),
        mesh=mesh,
        scratch_types=[
            pltpu.VMEM((CH_P, K), jnp.int32),
            pltpu.VMEM((CH_P, K), jnp.int32),
            pltpu.VMEM((K, QW), jnp.float32),
            pltpu.VMEM_SHARED((NPAD, QW), jnp.float32),
            pltpu.SemaphoreType.DMA,
        ],
    )


def _quarters(t):
    return jnp.stack([t[:, i * QW:(i + 1) * QW] for i in range(Q)], axis=0)


def _pre_body(deg_ref, x_ref, dis_ref, xs_ref):
    deg = deg_ref[0, :] + deg_ref[1, :] + 1.0
    dis = lax.rsqrt(deg)
    dis_ref[...] = dis[:, None]
    xs_ref[...] = _quarters(x_ref[...] * dis[:, None])


def _mid1_body(p_ref, u_ref, dis_ref, x_ref, n_ref,
               w1a_ref, w1b_ref, b1_ref, w2a_ref, w2b_ref, w2c_ref, out_ref):
    dis = dis_ref[...]
    t0 = (p_ref[0] + u_ref[0]) * dis
    t1 = (p_ref[1] + u_ref[1]) * dis
    h1 = jnp.dot(t0, w1a_ref[...], preferred_element_type=jnp.float32)
    h1 += jnp.dot(t1, w1b_ref[...], preferred_element_type=jnp.float32)
    h1 = jax.nn.relu(h1 + b1_ref[...])
    t2 = jnp.dot(x_ref[...], w2a_ref[...], preferred_element_type=jnp.float32)
    t2 += jnp.dot(n_ref[...], w2b_ref[...], preferred_element_type=jnp.float32)
    t2 += jnp.dot(h1, w2c_ref[...], preferred_element_type=jnp.float32)
    out_ref[...] = _quarters(t2 * dis)


def _mid2_body(p_ref, u_ref, dis_ref, x_ref,
               b2a_ref, b2b_ref, w3a_ref, w3c0_ref, w3c1_ref, out_ref):
    dis = dis_ref[...]
    h2_0 = jax.nn.relu((p_ref[0] + u_ref[0]) * dis + b2a_ref[...])
    h2_1 = jax.nn.relu((p_ref[1] + u_ref[1]) * dis + b2b_ref[...])
    t3 = jnp.dot(x_ref[...], w3a_ref[...], preferred_element_type=jnp.float32)
    t3 += jnp.dot(h2_0, w3c0_ref[...], preferred_element_type=jnp.float32)
    t3 += jnp.dot(h2_1, w3c1_ref[...], preferred_element_type=jnp.float32)
    out_ref[...] = _quarters(t3 * dis)


def _post_body(p_ref, u_ref, dis_ref, bm_ref, bl_ref, zm_ref, zl_ref):
    dis = dis_ref[...]
    zm_ref[...] = (p_ref[0] + u_ref[0]) * dis + bm_ref[...]
    zl_ref[...] = (p_ref[1] + u_ref[1]) * dis + bl_ref[...]


def _rowspec(width):
    return pl.BlockSpec((RB, width), lambda i: (i, 0))


def _qspec():
    return pl.BlockSpec((Q, RB, QW), lambda i: (0, i, 0))


def _fullspec(shape):
    nd = len(shape)
    return pl.BlockSpec(shape, lambda i, _nd=nd: (0,) * _nd)


_pre_call = pl.pallas_call(
    _pre_body,
    grid=(GRID,),
    in_specs=[pl.BlockSpec((NC, RB), lambda i: (0, i)), _rowspec(D_IN)],
    out_specs=[_rowspec(1), _qspec()],
    out_shape=[jax.ShapeDtypeStruct((N, 1), jnp.float32),
               jax.ShapeDtypeStruct((Q, N, QW), jnp.float32)],
)

_mid1_call = pl.pallas_call(
    _mid1_body,
    grid=(GRID,),
    in_specs=[_qspec(), _qspec(), _rowspec(1), _rowspec(D_IN),
              _rowspec(D_IN), _fullspec((QW, H1)), _fullspec((QW, H1)),
              _fullspec((1, H1)), _fullspec((D_IN, D_IN)),
              _fullspec((D_IN, D_IN)), _fullspec((H1, D_IN))],
    out_specs=_qspec(),
    out_shape=jax.ShapeDtypeStruct((Q, N, QW), jnp.float32),
)

_mid2_call = pl.pallas_call(
    _mid2_body,
    grid=(GRID,),
    in_specs=[_qspec(), _qspec(), _rowspec(1), _rowspec(D_IN),
              _fullspec((1, QW)), _fullspec((1, QW)),
              _fullspec((D_IN, D_IN)), _fullspec((QW, D_IN)),
              _fullspec((QW, D_IN))],
    out_specs=_qspec(),
    out_shape=jax.ShapeDtypeStruct((Q, N, QW), jnp.float32),
)

_post_call = pl.pallas_call(
    _post_body,
    grid=(GRID,),
    in_specs=[_qspec(), _qspec(), _rowspec(1),
              _fullspec((1, LAT)), _fullspec((1, LAT))],
    out_specs=[_rowspec(LAT), _rowspec(LAT)],
    out_shape=[jax.ShapeDtypeStruct((N, LAT), jnp.float32),
               jax.ShapeDtypeStruct((N, LAT), jnp.float32)],
)


def kernel(x, edge_index, noise, W1, b1, W2, b2, Wm, bm, Wl, bl):
    src = edge_index[0].astype(jnp.int32)
    dst = edge_index[1].astype(jnp.int32)

    dst_d = jnp.concatenate(
        [dst, jnp.full((EP_D - E,), DUMMY_D, jnp.int32)]).reshape(NC, NS, CH_D, KD)
    src_p = jnp.concatenate([src, jnp.zeros((EP_P - E,), jnp.int32)])
    src_p = jnp.stack([src_p, src_p + N]).reshape(NC, NS, CH_P, K)
    dst_p = jnp.concatenate(
        [dst, jnp.full((EP_P - E,), DUMMY, jnp.int32)]).reshape(NS, CH_P, K)

    zeros_1d = jnp.zeros((NPAD_D,), jnp.float32)
    ones_k = jnp.ones((KD,), jnp.float32)
    zeros_q = jnp.zeros((NPAD, QW), jnp.float32)

    w1a, w1b = W1[:QW], W1[QW:]
    w2a, w2b, w2c = W2[:D_IN], W2[D_IN:2 * D_IN], W2[2 * D_IN:]
    W3 = jnp.concatenate([Wm, Wl], axis=1)
    w3a, w3c0, w3c1 = W3[:D_IN], W3[D_IN:D_IN + QW], W3[D_IN + QW:]
    b1r = b1.reshape(1, H1)
    b2a, b2b = b2[:QW].reshape(1, QW), b2[QW:].reshape(1, QW)
    bmr, blr = bm.reshape(1, LAT), bl.reshape(1, LAT)

    deg2 = _deg_kernel()(dst_d, zeros_1d, ones_k)

    prop = _prop_kernel()
    dis, xs4 = _pre_call(deg2, x)
    p1 = prop(xs4.reshape(Q * N, QW), src_p, dst_p, zeros_q)
    t2s = _mid1_call(p1.reshape(Q, NPAD, QW), xs4, dis, x, noise,
                     w1a, w1b, b1r, w2a, w2b, w2c)
    p2 = prop(t2s.reshape(Q * N, QW), src_p, dst_p, zeros_q)
    t3s = _mid2_call(p2.reshape(Q, NPAD, QW), t2s, dis, x,
                     b2a, b2b, w3a, w3c0, w3c1)
    p3 = prop(t3s.reshape(Q * N, QW), src_p, dst_p, zeros_q)
    z_mean, z_logstd = _post_call(p3.reshape(Q, NPAD, QW), t3s, dis, bmr, blr)
    return (z_mean, z_logstd)
